# baseline (speedup 1.0000x reference)
import functools

import jax
import jax.numpy as jnp
from jax import lax
from jax.experimental import pallas as pl
from jax.experimental.pallas import tpu as pltpu
from jax.experimental.pallas import tpu_sc as plsc

B = 16384
D = 64
NC = 2
NS = 16
NW = NC * NS
BPW = B // NW
GROUPS = BPW // 16
LANE = 128
NCOLS = 7813
CPW = 250
CC = 2
NCH = CPW // CC
NBUF = 4
MAXBASE = NCOLS - CC


def _splat16(vec, idx16):
    return lax.gather(
        vec, idx16.reshape(16, 1),
        lax.GatherDimensionNumbers(
            offset_dims=(), collapsed_slice_dims=(0,),
            start_index_map=(0,)),
        slice_sizes=(1,),
        mode=lax.GatherScatterMode.PROMISE_IN_BOUNDS)


def _extract_body(su_hbm, upos_hbm, si_hbm, ipos_hbm, starts_hbm,
                  ut_hbm, it_hbm, sg_hbm,
                  ids_v, pos_v, colbuf_v, rowbufs_v, starts_v,
                  sema, semb, semc, semd, semw):
    wid = lax.axis_index("s") * NC + lax.axis_index("c")
    pltpu.sync_copy(starts_hbm, starts_v)
    lanes16 = lax.iota(jnp.int32, 16)
    dvecs = [lanes16 + 16 * c for c in range(D // 16)]
    wsplat = jnp.full((16,), wid, jnp.int32)
    col0 = wid * CPW
    sems = [sema, semb, semc, semd]

    def chunk_base(n):
        return pl.multiple_of(
            jnp.minimum(col0 + n * CC, MAXBASE) * LANE, LANE)

    for phase, (id_hbm, p_hbm, tab_hbm, off) in enumerate((
            (su_hbm, upos_hbm, ut_hbm, 0),
            (si_hbm, ipos_hbm, it_hbm, D))):
        pltpu.sync_copy(id_hbm, ids_v.at[pl.ds(0, B)])
        pltpu.sync_copy(p_hbm, pos_v.at[pl.ds(0, B)])
        sidx = wsplat + phase * (NW + 1)
        start_w = plsc.load_gather(starts_v, [sidx])[0]
        end_w = plsc.load_gather(starts_v, [sidx + 1])[0]
        ngroups = lax.div(end_w - start_w + 15, 16)

        def fire_chunk(n, slot):
            pltpu.async_copy(
                tab_hbm.at[:, pl.ds(chunk_base(n), CC * LANE)],
                colbuf_v.at[slot], sems[slot])

        def fire_chunk_dyn(n):
            for s in range(NBUF):
                @pl.when(n % NBUF == s)
                def _():
                    fire_chunk(n, s)

        def wait_chunk_dyn(n):
            for s in range(NBUF):
                @pl.when(n % NBUF == s)
                def _():
                    pltpu.make_async_copy(
                        tab_hbm.at[:, pl.ds(0, CC * LANE)],
                        colbuf_v.at[s], sems[s]).wait()

        for n in range(NBUF):
            fire_chunk(n, n)
        pltpu.make_async_copy(
            tab_hbm.at[:, pl.ds(0, CC * LANE)],
            colbuf_v.at[0], sems[0]).wait()

        def group(m, carry):
            c, prevfired, prevfired2 = carry
            gbase = start_w + m * 16
            ids16 = ids_v[pl.ds(gbase, 16)]
            pos16 = pos_v[pl.ds(gbase, 16)]
            nvalid = jnp.clip(end_w - gbase, 0, 16)
            for k in range(16):
                idk = ids16[k]
                posk = pos16[k]
                tc = lax.shift_right_logical(idk, 7)
                need = lax.div(tc - col0, CC)
                live = k < nvalid

                def adv_cond(cc_):
                    return jnp.logical_and(live, cc_ < need)

                def adv_body(cc_):
                    nxt = cc_ + NBUF

                    @pl.when(nxt < NCH)
                    def _():
                        fire_chunk_dyn(nxt)

                    wait_chunk_dyn(cc_ + 1)
                    return cc_ + 1

                c = lax.while_loop(adv_cond, adv_body, c)

                @pl.when(live)
                def _():
                    base = jnp.minimum(col0 + c * CC, MAXBASE) * LANE
                    lsplat = jnp.full((16,), idk - base, jnp.int32)
                    psplat = jnp.full((16,), c % NBUF, jnp.int32)
                    slot = (m % 3) * 16 + k
                    for cc4 in range(D // 16):
                        v = plsc.load_gather(
                            colbuf_v, [psplat, dvecs[cc4], lsplat])
                        rowbufs_v[pl.ds(slot * D + cc4 * 16, 16)] = v
                    pltpu.async_copy(
                        rowbufs_v.at[pl.ds(slot * D, D)],
                        sg_hbm.at[pl.ds(posk * (2 * D) + off, D)], semw)

            def drain(_, __):
                pltpu.make_async_copy(
                    sg_hbm.at[pl.ds(0, D)],
                    rowbufs_v.at[pl.ds(0, D)], semw).wait()
                return 0

            lax.fori_loop(0, prevfired2, drain, 0)
            return (c, nvalid, prevfired)

        c_fin, lastfired, lastfired2 = lax.fori_loop(
            0, ngroups, group, (jnp.int32(0), jnp.int32(0), jnp.int32(0)))

        def drain2(_, __):
            pltpu.make_async_copy(
                sg_hbm.at[pl.ds(0, D)],
                rowbufs_v.at[pl.ds(0, D)], semw).wait()
            return 0

        lax.fori_loop(0, lastfired + lastfired2, drain2, 0)
        for j in range(1, NBUF):
            @pl.when(c_fin + j < NCH)
            def _():
                wait_chunk_dyn(c_fin + j)


ASTRIDE = 17


def _reduce_body(sg_hbm, w_hbm, b_hbm, out_hbm, chunk_v, w_v, b_v, out_v,
                 acc_v, sem):
    wid = lax.axis_index("s") * NC + lax.axis_index("c")
    base = wid * BPW
    pltpu.sync_copy(w_hbm, w_v)
    pltpu.sync_copy(b_hbm, b_v)
    pltpu.async_copy(sg_hbm.at[pl.ds(base * (2 * D), BPW * 2 * D)],
                     chunk_v, sem).wait()

    bvec = b_v[...]
    wchunks = [w_v[pl.ds(c * 16, 16)] for c in range(D // 16)]
    lanes16 = lax.iota(jnp.int32, 16)

    def rowgroup(g, _):
        rb = g * 16
        for k in range(16):
            r = (rb + k) * (2 * D)
            acc = None
            for c in range(D // 16):
                u = chunk_v[pl.ds(r + c * 16, 16)]
                v = chunk_v[pl.ds(r + D + c * 16, 16)]
                p = u * v * wchunks[c]
                acc = p if acc is None else acc + p
            acc_v[pl.ds((rb + k) * ASTRIDE, 16)] = acc
        return 0

    lax.fori_loop(0, GROUPS, rowgroup, 0)

    def sumgroup(g, _):
        rows = (g * 16 + lanes16) * ASTRIDE
        acc = bvec
        for j in range(16):
            acc = acc + plsc.load_gather(acc_v, [rows + j])
        out_v[pl.ds(g * 16, 16)] = 1.0 / (1.0 + jnp.exp(-acc))
        return 0

    lax.fori_loop(0, GROUPS, sumgroup, 0)
    pltpu.sync_copy(out_v, out_hbm.at[pl.ds(base, BPW)])


@jax.jit
def _gmf_call(uid_flat, iid_flat, ut_t, it_t, w_flat, b_vec):
    mesh = plsc.VectorSubcoreMesh(core_axis_name="c", subcore_axis_name="s")
    cp = pltpu.CompilerParams(
        needs_layout_passes=False, use_tc_tiling_on_sc=True)

    pos_iota = lax.iota(jnp.int32, B)
    su, upos = lax.sort((uid_flat, pos_iota), num_keys=1)
    si, ipos = lax.sort((iid_flat, pos_iota), num_keys=1)
    bounds = jnp.arange(NW + 1, dtype=jnp.int32) * (CPW * LANE)
    us = jnp.searchsorted(su, bounds, side="left").astype(jnp.int32)
    is_ = jnp.searchsorted(si, bounds, side="left").astype(jnp.int32)
    starts = jnp.zeros((80,), jnp.int32)
    starts = starts.at[0:NW + 1].set(us)
    starts = starts.at[NW + 1:2 * NW + 2].set(is_)

    extract = functools.partial(
        pl.kernel,
        mesh=mesh,
        compiler_params=cp,
        out_type=jax.ShapeDtypeStruct((B * 2 * D,), jnp.float32),
        scratch_types=[
            pltpu.VMEM((B + 32,), jnp.int32),
            pltpu.VMEM((B + 32,), jnp.int32),
            pltpu.VMEM((NBUF, D, CC * LANE), jnp.float32),
            pltpu.VMEM((48 * D,), jnp.float32),
            pltpu.VMEM((80,), jnp.int32),
            pltpu.SemaphoreType.DMA,
            pltpu.SemaphoreType.DMA,
            pltpu.SemaphoreType.DMA,
            pltpu.SemaphoreType.DMA,
            pltpu.SemaphoreType.DMA,
        ],
    )(_extract_body)
    sg = extract(su, upos, si, ipos, starts, ut_t, it_t)

    reduce = functools.partial(
        pl.kernel,
        mesh=mesh,
        compiler_params=cp,
        out_type=jax.ShapeDtypeStruct((B,), jnp.float32),
        scratch_types=[
            pltpu.VMEM((BPW * 2 * D,), jnp.float32),
            pltpu.VMEM((D,), jnp.float32),
            pltpu.VMEM((16,), jnp.float32),
            pltpu.VMEM((BPW,), jnp.float32),
            pltpu.VMEM((BPW * ASTRIDE,), jnp.float32),
            pltpu.SemaphoreType.DMA,
        ],
    )(_reduce_body)
    return reduce(sg, w_flat, b_vec)


def kernel(user_ids, item_ids, user_table, item_table, W, b):
    uid_flat = user_ids.reshape(B).astype(jnp.int32)
    iid_flat = item_ids.reshape(B).astype(jnp.int32)
    ut_t = user_table.T
    it_t = item_table.T
    w_flat = W.reshape(D)
    b_vec = jnp.broadcast_to(b.reshape(1), (16,))
    out = _gmf_call(uid_flat, iid_flat, ut_t, it_t, w_flat, b_vec)
    return out.reshape(B, 1)


# speedup vs baseline: 1.0329x; 1.0329x over previous
import functools

import jax
import jax.numpy as jnp
from jax import lax
from jax.experimental import pallas as pl
from jax.experimental.pallas import tpu as pltpu
from jax.experimental.pallas import tpu_sc as plsc

B = 16384
D = 64
NC = 2
NS = 16
NW = NC * NS
BPW = B // NW
GROUPS = BPW // 16
LANE = 128
NCOLS = 7813
CPW = 250
CC = 5
NCH = CPW // CC
NBUF = 2
MAXBASE = NCOLS - CC


def _extract_body(su_hbm, upos_hbm, si_hbm, ipos_hbm, starts_hbm,
                  ut_hbm, it_hbm, sg_hbm,
                  ids_v, pos_v, colbuf_v, rowbufs_v, starts_v,
                  sema, semb, semc, semd, semw):
    wid = lax.axis_index("s") * NC + lax.axis_index("c")
    pltpu.sync_copy(starts_hbm, starts_v)
    lanes16 = lax.iota(jnp.int32, 16)
    dvecs = [lanes16 + 16 * c for c in range(D // 16)]
    wsplat = jnp.full((16,), wid, jnp.int32)
    col0 = wid * CPW
    sems = [sema, semb, semc, semd]

    def chunk_base(n):
        return pl.multiple_of(
            jnp.minimum(col0 + n * CC, MAXBASE) * LANE, LANE)

    for phase, (id_hbm, p_hbm, tab_hbm, off) in enumerate((
            (su_hbm, upos_hbm, ut_hbm, 0),
            (si_hbm, ipos_hbm, it_hbm, D))):
        pltpu.sync_copy(id_hbm, ids_v.at[pl.ds(0, B)])
        pltpu.sync_copy(p_hbm, pos_v.at[pl.ds(0, B)])
        sidx = wsplat + phase * (NW + 1)
        start_w = plsc.load_gather(starts_v, [sidx])[0]
        end_w = plsc.load_gather(starts_v, [sidx + 1])[0]
        ngroups = lax.div(end_w - start_w + 15, 16)

        def fire_chunk(n, slot):
            pltpu.async_copy(
                tab_hbm.at[:, pl.ds(chunk_base(n), CC * LANE)],
                colbuf_v.at[slot], sems[slot])

        def fire_chunk_dyn(n):
            for s in range(NBUF):
                @pl.when(n % NBUF == s)
                def _():
                    fire_chunk(n, s)

        def wait_chunk_dyn(n):
            for s in range(NBUF):
                @pl.when(n % NBUF == s)
                def _():
                    pltpu.make_async_copy(
                        tab_hbm.at[:, pl.ds(0, CC * LANE)],
                        colbuf_v.at[s], sems[s]).wait()

        for n in range(NBUF):
            fire_chunk(n, n)
        pltpu.make_async_copy(
            tab_hbm.at[:, pl.ds(0, CC * LANE)],
            colbuf_v.at[0], sems[0]).wait()

        def group(m, carry):
            c, prevfired, prevfired2 = carry
            gbase = start_w + m * 16
            ids16 = ids_v[pl.ds(gbase, 16)]
            pos16 = pos_v[pl.ds(gbase, 16)]
            nvalid = jnp.clip(end_w - gbase, 0, 16)
            for k in range(16):
                idk = ids16[k]
                posk = pos16[k]
                tc = lax.shift_right_logical(idk, 7)
                need = lax.div(tc - col0, CC)
                live = k < nvalid

                def adv_cond(cc_):
                    return jnp.logical_and(live, cc_ < need)

                def adv_body(cc_):
                    nxt = cc_ + NBUF

                    @pl.when(nxt < NCH)
                    def _():
                        fire_chunk_dyn(nxt)

                    wait_chunk_dyn(cc_ + 1)
                    return cc_ + 1

                c = lax.while_loop(adv_cond, adv_body, c)

                @pl.when(live)
                def _():
                    base = jnp.minimum(col0 + c * CC, MAXBASE) * LANE
                    lsplat = jnp.full((16,), idk - base, jnp.int32)
                    psplat = jnp.full((16,), c % NBUF, jnp.int32)
                    slot = (m % 3) * 16 + k
                    for cc4 in range(D // 16):
                        v = plsc.load_gather(
                            colbuf_v, [psplat, dvecs[cc4], lsplat])
                        rowbufs_v[pl.ds(slot * D + cc4 * 16, 16)] = v
                    pltpu.async_copy(
                        rowbufs_v.at[pl.ds(slot * D, D)],
                        sg_hbm.at[pl.ds(posk * (2 * D) + off, D)], semw)

            def drain(_, __):
                pltpu.make_async_copy(
                    sg_hbm.at[pl.ds(0, D)],
                    rowbufs_v.at[pl.ds(0, D)], semw).wait()
                return 0

            lax.fori_loop(0, prevfired2, drain, 0)
            return (c, nvalid, prevfired)

        c_fin, lastfired, lastfired2 = lax.fori_loop(
            0, ngroups, group, (jnp.int32(0), jnp.int32(0), jnp.int32(0)))

        def drain2(_, __):
            pltpu.make_async_copy(
                sg_hbm.at[pl.ds(0, D)],
                rowbufs_v.at[pl.ds(0, D)], semw).wait()
            return 0

        lax.fori_loop(0, lastfired + lastfired2, drain2, 0)
        for j in range(1, NBUF):
            @pl.when(c_fin + j < NCH)
            def _():
                wait_chunk_dyn(c_fin + j)


ASTRIDE = 17


def _reduce_body(sg_hbm, w_hbm, b_hbm, out_hbm, chunk_v, w_v, b_v, out_v,
                 acc_v, sem):
    wid = lax.axis_index("s") * NC + lax.axis_index("c")
    base = wid * BPW
    pltpu.sync_copy(w_hbm, w_v)
    pltpu.sync_copy(b_hbm, b_v)
    pltpu.async_copy(sg_hbm.at[pl.ds(base * (2 * D), BPW * 2 * D)],
                     chunk_v, sem).wait()

    bvec = b_v[...]
    wchunks = [w_v[pl.ds(c * 16, 16)] for c in range(D // 16)]
    lanes16 = lax.iota(jnp.int32, 16)

    def rowgroup(g, _):
        rb = g * 16
        for k in range(16):
            r = (rb + k) * (2 * D)
            acc = None
            for c in range(D // 16):
                u = chunk_v[pl.ds(r + c * 16, 16)]
                v = chunk_v[pl.ds(r + D + c * 16, 16)]
                p = u * v * wchunks[c]
                acc = p if acc is None else acc + p
            acc_v[pl.ds((rb + k) * ASTRIDE, 16)] = acc
        return 0

    lax.fori_loop(0, GROUPS, rowgroup, 0)

    def sumgroup(g, _):
        rows = (g * 16 + lanes16) * ASTRIDE
        acc = bvec
        for j in range(16):
            acc = acc + plsc.load_gather(acc_v, [rows + j])
        out_v[pl.ds(g * 16, 16)] = 1.0 / (1.0 + jnp.exp(-acc))
        return 0

    lax.fori_loop(0, GROUPS, sumgroup, 0)
    pltpu.sync_copy(out_v, out_hbm.at[pl.ds(base, BPW)])


@jax.jit
def _gmf_call(uid_flat, iid_flat, ut_t, it_t, w_flat, b_vec):
    mesh = plsc.VectorSubcoreMesh(core_axis_name="c", subcore_axis_name="s")
    cp = pltpu.CompilerParams(
        needs_layout_passes=False, use_tc_tiling_on_sc=True)

    pos_iota = lax.iota(jnp.int32, B)
    su, upos = lax.sort((uid_flat, pos_iota), num_keys=1)
    si, ipos = lax.sort((iid_flat, pos_iota), num_keys=1)
    bounds = jnp.arange(NW + 1, dtype=jnp.int32) * (CPW * LANE)
    us = jnp.searchsorted(su, bounds, side="left").astype(jnp.int32)
    is_ = jnp.searchsorted(si, bounds, side="left").astype(jnp.int32)
    starts = jnp.zeros((80,), jnp.int32)
    starts = starts.at[0:NW + 1].set(us)
    starts = starts.at[NW + 1:2 * NW + 2].set(is_)

    extract = functools.partial(
        pl.kernel,
        mesh=mesh,
        compiler_params=cp,
        out_type=jax.ShapeDtypeStruct((B * 2 * D,), jnp.float32),
        scratch_types=[
            pltpu.VMEM((B + 32,), jnp.int32),
            pltpu.VMEM((B + 32,), jnp.int32),
            pltpu.VMEM((NBUF, D, CC * LANE), jnp.float32),
            pltpu.VMEM((48 * D,), jnp.float32),
            pltpu.VMEM((80,), jnp.int32),
            pltpu.SemaphoreType.DMA,
            pltpu.SemaphoreType.DMA,
            pltpu.SemaphoreType.DMA,
            pltpu.SemaphoreType.DMA,
            pltpu.SemaphoreType.DMA,
        ],
    )(_extract_body)
    sg = extract(su, upos, si, ipos, starts, ut_t, it_t)

    reduce = functools.partial(
        pl.kernel,
        mesh=mesh,
        compiler_params=cp,
        out_type=jax.ShapeDtypeStruct((B,), jnp.float32),
        scratch_types=[
            pltpu.VMEM((BPW * 2 * D,), jnp.float32),
            pltpu.VMEM((D,), jnp.float32),
            pltpu.VMEM((16,), jnp.float32),
            pltpu.VMEM((BPW,), jnp.float32),
            pltpu.VMEM((BPW * ASTRIDE,), jnp.float32),
            pltpu.SemaphoreType.DMA,
        ],
    )(_reduce_body)
    return reduce(sg, w_flat, b_vec)


def kernel(user_ids, item_ids, user_table, item_table, W, b):
    uid_flat = user_ids.reshape(B).astype(jnp.int32)
    iid_flat = item_ids.reshape(B).astype(jnp.int32)
    ut_t = user_table.T
    it_t = item_table.T
    w_flat = W.reshape(D)
    b_vec = jnp.broadcast_to(b.reshape(1), (16,))
    out = _gmf_call(uid_flat, iid_flat, ut_t, it_t, w_flat, b_vec)
    return out.reshape(B, 1)
